# baseline (speedup 1.0000x reference)
import functools

import jax
import jax.numpy as jnp
from jax import lax
from jax.experimental import pallas as pl
from jax.experimental.pallas import tpu as pltpu
from jax.experimental.pallas import tpu_sc as plsc

N = 10000
E = 320000
D = 128

NC = 2
NS = 16
NW = NC * NS

CHUNK = 120
EW = E // NW
NB = 3
NCHUNK = 84
EWP = NCHUNK * CHUNK
TCHUNK = NCHUNK + NB
NP = 10112
ROWS_PER_TILE = NP // NS

def _sc_agg_body(idx_hbm, h_hbm, zrows_hbm, agg_out,
                 idxs, bufs, isems, gsems, acc_sh):
    cid = lax.axis_index("c")
    sid = lax.axis_index("s")
    wid = sid * NC + cid
    base = sid * ROWS_PER_TILE

    pltpu.sync_copy(zrows_hbm, acc_sh.at[pl.ds(base, ROWS_PER_TILE)])

    def fire_idx(k, b):
        pltpu.async_copy(idx_hbm.at[wid, k], idxs[b], isems[b])

    def wait_idx(k, b):
        pltpu.make_async_copy(idx_hbm.at[wid, k], idxs[b], isems[b]).wait()

    def fire_gather(k, b):
        pltpu.async_copy(h_hbm.at[idxs[b].at[0]], bufs[b], gsems[b])

    def wait_gather(k, b):
        pltpu.make_async_copy(h_hbm.at[idxs[b].at[0]], bufs[b],
                              gsems[b]).wait()

    def scatter(k, b):
        pltpu.sync_copy(bufs[b], acc_sh.at[idxs[b].at[1]], add=True)

    plsc.subcore_barrier()

    for b in range(NB):
        fire_idx(b, b)
    for b in range(NB - 1):
        wait_idx(b, b)
        fire_gather(b, b)

    @pl.loop(0, NCHUNK // NB)
    def _groups(i):
        g = i * NB
        for b in range(NB):
            k = g + b
            kg = k + NB - 1
            bg = (b + NB - 1) % NB
            wait_idx(kg, bg)
            fire_gather(kg, bg)
            wait_gather(k, b)
            scatter(k, b)
            fire_idx(k + NB, b)

    for j in range(NB - 1):
        k = NCHUNK + j
        wait_gather(k, k % NB)
    k = NCHUNK + NB - 1
    wait_idx(k, k % NB)

    plsc.subcore_barrier()

    pltpu.sync_copy(acc_sh.at[pl.ds(base, ROWS_PER_TILE)],
                    agg_out.at[cid, pl.ds(base, ROWS_PER_TILE)])


@functools.cache
def _sc_kernels():
    mesh = plsc.VectorSubcoreMesh(core_axis_name="c", subcore_axis_name="s",
                                  num_cores=NC, num_subcores=NS)
    sc_agg = pl.kernel(
        _sc_agg_body,
        out_type=jax.ShapeDtypeStruct((NC, NP, D), jnp.float32),
        mesh=mesh,
        scratch_types=[
            [pltpu.VMEM((2, CHUNK), jnp.int32)] * NB,
            [pltpu.VMEM((CHUNK, D), jnp.float32)] * NB,
            [pltpu.SemaphoreType.DMA] * NB,
            [pltpu.SemaphoreType.DMA] * NB,
            pltpu.VMEM_SHARED((NP, D), jnp.float32),
        ],
    )
    return sc_agg


BLK = 400
GRID = N // BLK


def _dense_body(apply_relu, h_ref, agg_ref, deg_ref, w_ref, b_ref, o_ref):
    deg = deg_ref[0, :, 0:1] + deg_ref[1, :, 0:1]
    denom = jnp.maximum(deg, 1.0)
    agg = (agg_ref[0] + agg_ref[1]) / denom
    h = h_ref[...]
    w = w_ref[...]
    x = lax.dot_general(h, w[:, :D], (((1,), (1,)), ((), ())),
                        preferred_element_type=jnp.float32)
    x = x + lax.dot_general(agg, w[:, D:], (((1,), (1,)), ((), ())),
                            preferred_element_type=jnp.float32)
    x = x + b_ref[...]
    if apply_relu:
        x = jnp.where(x > 0, x, 0.01 * x)
    nrm = jnp.sqrt(jnp.sum(x * x, axis=1, keepdims=True))
    o_ref[...] = x / jnp.maximum(nrm, 1e-6)


def _dense(h, aggP, degP, W, b, apply_relu):
    return pl.pallas_call(
        functools.partial(_dense_body, apply_relu),
        grid=(GRID,),
        in_specs=[
            pl.BlockSpec((BLK, D), lambda i: (i, 0)),
            pl.BlockSpec((NC, BLK, D), lambda i: (0, i, 0)),
            pl.BlockSpec((NC, BLK, D), lambda i: (0, i, 0)),
            pl.BlockSpec((D, 2 * D), lambda i: (0, 0)),
            pl.BlockSpec((1, D), lambda i: (0, 0)),
        ],
        out_specs=pl.BlockSpec((BLK, D), lambda i: (i, 0)),
        out_shape=jax.ShapeDtypeStruct((N, D), jnp.float32),
    )(h, aggP, degP, W, b)


def kernel(edge_index, node_emb, W0, b0, W1, b1, W2, b2):
    pad = TCHUNK * CHUNK - EW
    pad_src = jnp.zeros((NW, pad), jnp.int32)
    pad_dst = jnp.full((NW, pad), N, jnp.int32)
    src = jnp.concatenate([edge_index[0].reshape(NW, EW), pad_src],
                          axis=1).reshape(NW, TCHUNK, CHUNK)
    dst = jnp.concatenate([edge_index[1].reshape(NW, EW), pad_dst],
                          axis=1).reshape(NW, TCHUNK, CHUNK)
    idx_tab = jnp.stack([src, dst], axis=2)
    h = node_emb[1:]
    zrows = jnp.zeros((ROWS_PER_TILE, D), jnp.float32)
    ones_nd = jnp.ones((N, D), jnp.float32)

    _sc_agg = _sc_kernels()
    degP = _sc_agg(idx_tab, ones_nd, zrows)
    aggP = _sc_agg(idx_tab, h, zrows)
    h = _dense(h, aggP, degP, W0, b0.reshape(1, D), True)
    aggP = _sc_agg(idx_tab, h, zrows)
    h = _dense(h, aggP, degP, W1, b1.reshape(1, D), True)
    aggP = _sc_agg(idx_tab, h, zrows)
    h = _dense(h, aggP, degP, W2, b2.reshape(1, D), False)
    return h
